# baseline (speedup 1.0000x reference)
import functools
import jax
import jax.numpy as jnp
from jax.experimental import pallas as pl
from jax.experimental.pallas import tpu as pltpu

_DEPTH = 1000
_BATCH = 16384
_BLK = 1024
_NCHUNK = _BATCH // _BLK
_NBUF = 4


def _tc_body(idx_ref, out_ref, *scr):
    bufs = scr[:_NBUF]
    sems = scr[_NBUF:]
    cols = jax.lax.broadcasted_iota(jnp.int32, (_BLK, _DEPTH), 1)
    copies = [None] * _NBUF
    for c in range(_NCHUNK):
        b = c % _NBUF
        if copies[b] is not None:
            copies[b].wait()
        idx = idx_ref[pl.ds(c * _BLK, _BLK), :]
        bufs[b][...] = (cols == idx).astype(jnp.float32)
        copies[b] = pltpu.make_async_copy(
            bufs[b], out_ref.at[pl.ds(c * _BLK, _BLK), :], sems[b])
        copies[b].start()
    for b in range(_NBUF):
        copies[b].wait()


@jax.jit
def _tc_onehot(idx2):
    return pl.pallas_call(
        _tc_body,
        in_specs=[pl.BlockSpec(memory_space=pltpu.VMEM)],
        out_specs=pl.BlockSpec(memory_space=pl.ANY),
        out_shape=jax.ShapeDtypeStruct((_BATCH, _DEPTH), jnp.float32),
        scratch_shapes=(
            [pltpu.VMEM((_BLK, _DEPTH), jnp.float32) for _ in range(_NBUF)]
            + [pltpu.SemaphoreType.DMA for _ in range(_NBUF)]
        ),
    )(idx2)


def kernel(X_in, ones):
    del ones
    idx = X_in.astype(jnp.int32).reshape(_BATCH, 1)
    return _tc_onehot(idx)
